# baseline (speedup 1.0000x reference)
import functools

import jax
import jax.numpy as jnp
from jax import lax
from jax.experimental import pallas as pl
from jax.experimental.pallas import tpu as pltpu
from jax.experimental.pallas import tpu_sc as plsc

NUM_NODES = 1000000
NODE_DIM = 64
PATH_LEN = 16384

NC = 2
NS = 16
NW = NC * NS
BPW = PATH_LEN // NW
CHUNK = 128
NCHUNK = BPW // CHUNK


def _body(path_hbm, mask_hbm, table_hbm, zo_hbm, out_hbm,
          idx_v, mask_v, rows_v, mexp_v, sem):
    wid = lax.axis_index("s") * NC + lax.axis_index("c")
    base = wid * BPW

    for j in range(NCHUNK):
        pltpu.sync_copy(path_hbm.at[pl.ds(base + j * CHUNK, CHUNK)], idx_v.at[j])
        pltpu.sync_copy(mask_hbm.at[pl.ds(base + j * CHUNK, CHUNK)], mask_v.at[j])

    for j in range(NCHUNK):
        for k in range(CHUNK // 16):
            v = mask_v[j, pl.ds(k * 16, 16)]
            mask_v[j, pl.ds(k * 16, 16)] = jnp.minimum(jnp.maximum(v, 0), 1)

    copies = []
    for j in range(NCHUNK):
        copies.append(
            pltpu.async_copy(
                table_hbm.at[idx_v.at[j]],
                rows_v.at[pl.ds(j * CHUNK, CHUNK)],
                sem,
            )
        )
        copies.append(
            pltpu.async_copy(
                zo_hbm.at[mask_v.at[j]],
                mexp_v.at[pl.ds(j * CHUNK, CHUNK)],
                sem,
            )
        )
    for c in copies:
        c.wait()

    def mask_row(r, _):
        for c in range(NODE_DIM // 16):
            sl = pl.ds(c * 16, 16)
            rows_v[r, sl] = rows_v[r, sl] * mexp_v[r, sl]
        return 0

    lax.fori_loop(0, BPW, mask_row, 0)

    pltpu.sync_copy(rows_v, out_hbm.at[pl.ds(base, BPW)])


def kernel(path, mask, table):
    mesh = plsc.VectorSubcoreMesh(core_axis_name="c", subcore_axis_name="s")
    f = functools.partial(
        pl.kernel,
        mesh=mesh,
        compiler_params=pltpu.CompilerParams(use_tc_tiling_on_sc=False),
        out_type=jax.ShapeDtypeStruct((PATH_LEN, NODE_DIM), jnp.float32),
        scratch_types=[
            pltpu.VMEM((NCHUNK, CHUNK), jnp.int32),
            pltpu.VMEM((NCHUNK, CHUNK), jnp.int32),
            pltpu.VMEM((BPW, NODE_DIM), jnp.float32),
            pltpu.VMEM((BPW, NODE_DIM), jnp.float32),
            pltpu.SemaphoreType.DMA,
        ],
    )(_body)
    zo = jnp.zeros((8, NODE_DIM), jnp.float32).at[1].set(1.0)
    return f(path.astype(jnp.int32), mask, table, zo)


# speedup vs baseline: 1.4872x; 1.4872x over previous
import functools

import jax
import jax.numpy as jnp
from jax import lax
from jax.experimental import pallas as pl
from jax.experimental.pallas import tpu as pltpu
from jax.experimental.pallas import tpu_sc as plsc

NUM_NODES = 1000000
NODE_DIM = 64
PATH_LEN = 16384

NC = 2
NS = 16
NW = NC * NS
BPW = PATH_LEN // NW
CHUNK = 128
NCHUNK = BPW // CHUNK


def _body(path_hbm, mask_hbm, table_hbm, out_hbm,
          idx_v, mask_v, maskf_v, rows_v, sem):
    wid = lax.axis_index("s") * NC + lax.axis_index("c")
    base = wid * BPW

    for j in range(NCHUNK):
        pltpu.sync_copy(path_hbm.at[pl.ds(base + j * CHUNK, CHUNK)], idx_v.at[j])
        pltpu.sync_copy(mask_hbm.at[pl.ds(base + j * CHUNK, CHUNK)], mask_v.at[j])

    copies = []
    for j in range(NCHUNK):
        copies.append(
            pltpu.async_copy(
                table_hbm.at[idx_v.at[j]],
                rows_v.at[pl.ds(j * CHUNK, CHUNK)],
                sem,
            )
        )

    for j in range(NCHUNK):
        for k in range(CHUNK // 16):
            v = mask_v[j, pl.ds(k * 16, 16)]
            v01 = jnp.minimum(jnp.maximum(v, 0), 1)
            maskf_v[pl.ds(j * CHUNK + k * 16, 16)] = v01.astype(jnp.float32)

    for c in copies:
        c.wait()

    def mask_row(r, _):
        mf = plsc.load_gather(maskf_v, [jnp.broadcast_to(r, (16,))])
        for c in range(NODE_DIM // 16):
            sl = pl.ds(c * 16, 16)
            rows_v[r, sl] = rows_v[r, sl] * mf
        return 0

    lax.fori_loop(0, BPW, mask_row, 0)

    pltpu.sync_copy(rows_v, out_hbm.at[pl.ds(base, BPW)])


def kernel(path, mask, table):
    mesh = plsc.VectorSubcoreMesh(core_axis_name="c", subcore_axis_name="s")
    f = functools.partial(
        pl.kernel,
        mesh=mesh,
        compiler_params=pltpu.CompilerParams(
            use_tc_tiling_on_sc=False,
            needs_layout_passes=False,
        ),
        out_type=jax.ShapeDtypeStruct((PATH_LEN, NODE_DIM), jnp.float32),
        scratch_types=[
            pltpu.VMEM((NCHUNK, CHUNK), jnp.int32),
            pltpu.VMEM((NCHUNK, CHUNK), jnp.int32),
            pltpu.VMEM((BPW,), jnp.float32),
            pltpu.VMEM((BPW, NODE_DIM), jnp.float32),
            pltpu.SemaphoreType.DMA,
        ],
    )(_body)
    return f(path.astype(jnp.int32), mask, table)
